# baseline (speedup 1.0000x reference)
import jax
import jax.numpy as jnp
from jax import lax
from jax.experimental import pallas as pl
from jax.experimental.pallas import tpu as pltpu

_V = 100000
_D = 640
_NCTX = 20
_ED = 32
_TILE = 10000
_NT = _V // _TILE


def _gather_body(idx_smem, emb_hbm, o_hbm, sem):
    copies = []
    for k in range(_NCTX):
        row = idx_smem[k]
        cp = pltpu.make_async_copy(
            emb_hbm.at[pl.ds(row, 1), :],
            o_hbm.at[pl.ds(k, 1), :],
            sem,
        )
        cp.start()
        copies.append(cp)
    for cp in copies:
        cp.wait()


def _gather(inputs, emb):
    grid_spec = pltpu.PrefetchScalarGridSpec(
        num_scalar_prefetch=1,
        grid=(1,),
        in_specs=[pl.BlockSpec(memory_space=pl.ANY)],
        out_specs=pl.BlockSpec(memory_space=pl.ANY),
        scratch_shapes=[pltpu.SemaphoreType.DMA],
    )
    return pl.pallas_call(
        _gather_body,
        grid_spec=grid_spec,
        out_shape=jax.ShapeDtypeStruct((_NCTX, _ED), jnp.float32),
    )(inputs, emb)


_H = _TILE // 2


def _mv_body(e_ref, wa_ref, wb_ref, b_ref, out_ref, acc_ref):
    i = pl.program_id(0)

    @pl.when(i == 0)
    def _():
        acc_ref[0] = -jnp.inf
        acc_ref[1] = 0.0

    dn = (((1,), (1,)), ((), ()))
    ta = lax.dot_general(e_ref[...], wa_ref[...], dn,
                         preferred_element_type=jnp.float32)
    tb = lax.dot_general(e_ref[...], wb_ref[...], dn,
                         preferred_element_type=jnp.float32)
    ta = ta + b_ref[0, :, 0:_H]
    tb = tb + b_ref[0, :, _H:_TILE]
    out_ref[pl.ds(i, 1), 0:_H] = ta
    out_ref[pl.ds(i, 1), _H:_TILE] = tb

    m_prev = acc_ref[0]
    s_prev = acc_ref[1]
    m_new = jnp.maximum(m_prev,
                        jnp.maximum(jnp.max(ta), jnp.max(tb)))
    s_new = (s_prev * jnp.exp(m_prev - m_new)
             + jnp.sum(jnp.exp(ta - m_new))
             + jnp.sum(jnp.exp(tb - m_new)))
    acc_ref[0] = m_new
    acc_ref[1] = s_new

    @pl.when(i == _NT - 1)
    def _():
        out_ref[...] = out_ref[...] - (m_new + jnp.log(s_new))


def _matvec_logsoftmax(e, W1, b1r):
    return pl.pallas_call(
        _mv_body,
        grid=(_NT,),
        in_specs=[
            pl.BlockSpec((1, _D), lambda i: (0, 0)),
            pl.BlockSpec((_H, _D), lambda i: (2 * i, 0)),
            pl.BlockSpec((_H, _D), lambda i: (2 * i + 1, 0)),
            pl.BlockSpec((1, 1, _TILE), lambda i: (i, 0, 0)),
        ],
        out_specs=pl.BlockSpec((_NT, _TILE), lambda i: (0, 0)),
        out_shape=jax.ShapeDtypeStruct((_NT, _TILE), jnp.float32),
        scratch_shapes=[pltpu.SMEM((2,), jnp.float32)],
    )(e, W1, W1, b1r)


def kernel(inputs, emb, W1, b1):
    rows = _gather(inputs, emb)
    e = rows.reshape(1, _D)
    b1r = b1.reshape(_NT, 1, _TILE)
    log_probs = _matvec_logsoftmax(e, W1, b1r)
    return log_probs.reshape(1, _V)


# speedup vs baseline: 1.0244x; 1.0244x over previous
import jax
import jax.numpy as jnp
from jax import lax
from jax.experimental import pallas as pl
from jax.experimental.pallas import tpu as pltpu

_V = 100000
_D = 640
_NCTX = 20
_ED = 32
_TILE = 4000
_NT = _V // _TILE


def _gather_body(idx_smem, emb_hbm, o_hbm, sem):
    copies = []
    for k in range(_NCTX):
        row = idx_smem[k]
        cp = pltpu.make_async_copy(
            emb_hbm.at[pl.ds(row, 1), :],
            o_hbm.at[pl.ds(k, 1), :],
            sem,
        )
        cp.start()
        copies.append(cp)
    for cp in copies:
        cp.wait()


def _gather(inputs, emb):
    grid_spec = pltpu.PrefetchScalarGridSpec(
        num_scalar_prefetch=1,
        grid=(1,),
        in_specs=[pl.BlockSpec(memory_space=pl.ANY)],
        out_specs=pl.BlockSpec(memory_space=pl.ANY),
        scratch_shapes=[pltpu.SemaphoreType.DMA],
    )
    return pl.pallas_call(
        _gather_body,
        grid_spec=grid_spec,
        out_shape=jax.ShapeDtypeStruct((_NCTX, _ED), jnp.float32),
    )(inputs, emb)


_H = _TILE // 2


def _mv_body(er_ref, wa_ref, wb_ref, b_ref, out_ref, e_ref, acc_ref):
    i = pl.program_id(0)

    @pl.when(i == 0)
    def _():
        acc_ref[0] = -jnp.inf
        acc_ref[1] = 0.0
        e_ref[...] = jnp.concatenate(
            [er_ref[k:k + 1, :] for k in range(_NCTX)], axis=1)

    dn = (((1,), (1,)), ((), ()))
    ta = lax.dot_general(e_ref[...], wa_ref[...], dn,
                         preferred_element_type=jnp.float32)
    tb = lax.dot_general(e_ref[...], wb_ref[...], dn,
                         preferred_element_type=jnp.float32)
    ta = ta + b_ref[0, :, 0:_H]
    tb = tb + b_ref[0, :, _H:_TILE]
    out_ref[pl.ds(i, 1), 0:_H] = ta
    out_ref[pl.ds(i, 1), _H:_TILE] = tb

    m_prev = acc_ref[0]
    s_prev = acc_ref[1]
    m_new = jnp.maximum(m_prev,
                        jnp.maximum(jnp.max(ta), jnp.max(tb)))
    s_new = (s_prev * jnp.exp(m_prev - m_new)
             + jnp.sum(jnp.exp(ta - m_new))
             + jnp.sum(jnp.exp(tb - m_new)))
    acc_ref[0] = m_new
    acc_ref[1] = s_new

    @pl.when(i == _NT - 1)
    def _():
        out_ref[...] = out_ref[...] - (m_new + jnp.log(s_new))


def _matvec_logsoftmax(e, W1, b1r):
    return pl.pallas_call(
        _mv_body,
        grid=(_NT,),
        in_specs=[
            pl.BlockSpec((_NCTX, _ED), lambda i: (0, 0)),
            pl.BlockSpec((_H, _D), lambda i: (2 * i, 0)),
            pl.BlockSpec((_H, _D), lambda i: (2 * i + 1, 0)),
            pl.BlockSpec((1, 1, _TILE), lambda i: (i, 0, 0)),
        ],
        out_specs=pl.BlockSpec((_NT, _TILE), lambda i: (0, 0)),
        out_shape=jax.ShapeDtypeStruct((_NT, _TILE), jnp.float32),
        scratch_shapes=[pltpu.VMEM((1, _D), jnp.float32),
                        pltpu.SMEM((2,), jnp.float32)],
    )(e, W1, W1, b1r)


def kernel(inputs, emb, W1, b1):
    e = _gather(inputs, emb)
    b1r = b1.reshape(_NT, 1, _TILE)
    log_probs = _matvec_logsoftmax(e, W1, b1r)
    return log_probs.reshape(1, _V)
